# baseline (speedup 1.0000x reference)
import jax
import jax.numpy as jnp
from jax import lax
from jax.experimental import pallas as pl
from jax.experimental.pallas import tpu as pltpu
from jax.experimental.pallas import tpu_sc as plsc

B = 16384
EMB = 128
LAT = 64

NC, NS = 2, 16
NW = NC * NS
NSPLIT = 2
BSPLIT = B // NSPLIT
ROWS_PER_W = BSPLIT // NW
CHUNK = 128
NCH = ROWS_PER_W // CHUNK
NCHUNKS = 3 * NCH

BM = 2048
NB = BSPLIT // BM


def _gather_body(gidx, bidx, pidx, gt, bt, pt, o1, o2, o3,
                 idx_v, rows_v, *sems):
    gsems, wsems = sems[:NCHUNKS], sems[NCHUNKS:]
    wid = lax.axis_index("s") * NC + lax.axis_index("c")
    base = wid * ROWS_PER_W
    tbls = (gt, bt, pt)
    outs = (o1, o2, o3)

    for t, idx_hbm in enumerate((gidx, bidx, pidx)):
        pltpu.sync_copy(idx_hbm.at[wid], idx_v.at[pl.ds(t * NCH, NCH)])

    gdescs = [
        pltpu.async_copy(tbls[c // NCH].at[idx_v.at[c]], rows_v.at[c],
                         gsems[c])
        for c in range(NCHUNKS)
    ]
    wdescs = []
    for c in range(NCHUNKS):
        gdescs[c].wait()
        t, j = divmod(c, NCH)
        wdescs.append(pltpu.async_copy(
            rows_v.at[c], outs[t].at[pl.ds(base + j * CHUNK, CHUNK)],
            wsems[c]))
    for w in wdescs:
        w.wait()


def _mlp_body(acc, g, bw, p, wp, bp, wf, bfc, o):
    wfv = wf[...]
    m1 = jnp.dot(wp[0 * EMB:1 * EMB, :], wfv,
                 preferred_element_type=jnp.float32)
    m2 = jnp.dot(wp[1 * EMB:2 * EMB, :], wfv,
                 preferred_element_type=jnp.float32)
    m3 = jnp.dot(wp[2 * EMB:3 * EMB, :], wfv,
                 preferred_element_type=jnp.float32)
    dn = (((0,), (1,)), ((), ()))
    r = lax.dot_general(m1, g[...], dn, preferred_element_type=jnp.float32)
    r += lax.dot_general(m2, bw[...], dn, preferred_element_type=jnp.float32)
    r += lax.dot_general(m3, p[...], dn, preferred_element_type=jnp.float32)
    ct = lax.dot_general(wfv, bp[...], dn,
                         preferred_element_type=jnp.float32)
    o[...] = r + (ct + bfc[...].reshape(LAT, 1))


def kernel(gains, bws, pms, gain_table, bw_table, pm_table,
           W_proj, b_proj, W_fc, b_fc):
    gi = gains.astype(jnp.int32).reshape(NSPLIT, NW, NCH, CHUNK)
    bi = bws.astype(jnp.int32).reshape(NSPLIT, NW, NCH, CHUNK)
    pi = pms.astype(jnp.int32).reshape(NSPLIT, NW, NCH, CHUNK)

    gather = pl.kernel(
        _gather_body,
        mesh=plsc.VectorSubcoreMesh(core_axis_name="c", subcore_axis_name="s"),
        out_type=[jax.ShapeDtypeStruct((BSPLIT, EMB), jnp.float32)] * 3,
        scratch_types=[
            pltpu.VMEM((NCHUNKS, CHUNK), jnp.int32),
            pltpu.VMEM((NCHUNKS, CHUNK, EMB), jnp.float32),
        ] + [pltpu.SemaphoreType.DMA] * (2 * NCHUNKS),
    )

    bp2 = b_proj.reshape(1, EMB)
    bf2 = b_fc.reshape(1, LAT)
    row_spec = pl.BlockSpec((BM, EMB), lambda i: (i, 0))
    w_specs = [
        pl.BlockSpec((3 * EMB, EMB), lambda i: (0, 0)),
        pl.BlockSpec((1, EMB), lambda i: (0, 0)),
        pl.BlockSpec((EMB, LAT), lambda i: (0, 0)),
        pl.BlockSpec((1, LAT), lambda i: (0, 0)),
    ]

    out_t = None
    for k in range(NSPLIT):
        ge, be, pe = gather(gi[k], bi[k], pi[k],
                            gain_table, bw_table, pm_table)
        if out_t is None:
            acc_ops = []
            acc_specs = []
            alias = {}
        else:
            acc_ops = [out_t]
            acc_specs = [pl.BlockSpec((LAT, BM), lambda i: (0, 0))]
            alias = {0: 0}
        out_t = pl.pallas_call(
            _mlp_body if acc_ops else _mlp_first_body,
            grid=(NB,),
            in_specs=acc_specs + [row_spec, row_spec, row_spec] + w_specs,
            out_specs=pl.BlockSpec((LAT, BM), lambda i, k=k: (0, k * NB + i)),
            out_shape=jax.ShapeDtypeStruct((LAT, B), jnp.float32),
            input_output_aliases=alias,
        )(*acc_ops, ge, be, pe, W_proj, bp2, W_fc, bf2)
    return out_t.T


def _mlp_first_body(g, bw, p, wp, bp, wf, bfc, o):
    _mlp_body(None, g, bw, p, wp, bp, wf, bfc, o)


# speedup vs baseline: 1.0479x; 1.0479x over previous
import jax
import jax.numpy as jnp
from jax import lax
from jax.experimental import pallas as pl
from jax.experimental.pallas import tpu as pltpu
from jax.experimental.pallas import tpu_sc as plsc

B = 16384
EMB = 128
LAT = 64

NC, NS = 2, 16
NW = NC * NS
CHUNK = 128
UNIT = NW * CHUNK
NUNITS = B // UNIT

SPLITS = (12288, 4096)
BM = 2048


def _make_gather_body(off, nch, slots, window):
    total = 3 * nch
    u0 = off // UNIT

    def body(gidx, bidx, pidx, gt, bt, pt, o1, o2, o3, idx_v, rows_v, *sems):
        gsems, wsems = sems[:slots], sems[slots:]
        wid = lax.axis_index("s") * NC + lax.axis_index("c")
        tbls = (gt, bt, pt)
        outs = (o1, o2, o3)

        sdescs = []
        for t, ih in enumerate((gidx, bidx, pidx)):
            for uu in range(nch):
                c = t * nch + uu
                sdescs.append(pltpu.async_copy(
                    ih.at[u0 + uu, wid], idx_v.at[pl.ds(c, 1)],
                    wsems[c % slots]))
        for d in sdescs:
            d.wait()

        gdescs = [None] * total
        wdescs = [None] * total

        def fire(c):
            slot = c % slots
            if c >= slots:
                wdescs[c - slots].wait()
            gdescs[c] = pltpu.async_copy(
                tbls[c // nch].at[idx_v.at[c]], rows_v.at[slot], gsems[slot])

        for c in range(min(window, total)):
            fire(c)
        for c in range(total):
            gdescs[c].wait()
            t, uu = divmod(c, nch)
            slot = c % slots
            wdescs[c] = pltpu.async_copy(
                rows_v.at[slot],
                outs[t].at[pl.ds(uu * UNIT + wid * CHUNK, CHUNK)],
                wsems[slot])
            if c + window < total:
                fire(c + window)
        for c in range(max(0, total - slots), total):
            if wdescs[c] is not None:
                wdescs[c].wait()

    return body


def _mlp_math(g, bw, p, wp, bp, wf, bfc, o):
    wfv = wf[...]
    m1 = jnp.dot(wp[0 * EMB:1 * EMB, :], wfv,
                 preferred_element_type=jnp.float32)
    m2 = jnp.dot(wp[1 * EMB:2 * EMB, :], wfv,
                 preferred_element_type=jnp.float32)
    m3 = jnp.dot(wp[2 * EMB:3 * EMB, :], wfv,
                 preferred_element_type=jnp.float32)
    dn = (((0,), (1,)), ((), ()))
    r = lax.dot_general(m1, g[...], dn, preferred_element_type=jnp.float32)
    r += lax.dot_general(m2, bw[...], dn, preferred_element_type=jnp.float32)
    r += lax.dot_general(m3, p[...], dn, preferred_element_type=jnp.float32)
    ct = lax.dot_general(wfv, bp[...], dn,
                         preferred_element_type=jnp.float32)
    o[...] = r + (ct + bfc[...].reshape(LAT, 1))


def _mlp_first(g, bw, p, wp, bp, wf, bfc, o):
    _mlp_math(g, bw, p, wp, bp, wf, bfc, o)


def _mlp_acc(acc, g, bw, p, wp, bp, wf, bfc, o):
    _mlp_math(g, bw, p, wp, bp, wf, bfc, o)


def kernel(gains, bws, pms, gain_table, bw_table, pm_table,
           W_proj, b_proj, W_fc, b_fc):
    gi = gains.astype(jnp.int32).reshape(NUNITS, NW, 1, CHUNK)
    bi = bws.astype(jnp.int32).reshape(NUNITS, NW, 1, CHUNK)
    pi = pms.astype(jnp.int32).reshape(NUNITS, NW, 1, CHUNK)

    bp2 = b_proj.reshape(1, EMB)
    bf2 = b_fc.reshape(1, LAT)
    row_spec = pl.BlockSpec((BM, EMB), lambda i: (i, 0))
    w_specs = [
        pl.BlockSpec((3 * EMB, EMB), lambda i: (0, 0)),
        pl.BlockSpec((1, EMB), lambda i: (0, 0)),
        pl.BlockSpec((EMB, LAT), lambda i: (0, 0)),
        pl.BlockSpec((1, LAT), lambda i: (0, 0)),
    ]
    mesh = plsc.VectorSubcoreMesh(core_axis_name="c", subcore_axis_name="s")

    out_t = None
    off = 0
    for bs in SPLITS:
        nch = bs // NW // CHUNK
        total = 3 * nch
        slots = min(total, 7)
        window = min(total, 4)
        gather = pl.kernel(
            _make_gather_body(off, nch, slots, window),
            mesh=mesh,
            out_type=[jax.ShapeDtypeStruct((bs, EMB), jnp.float32)] * 3,
            scratch_types=[
                pltpu.VMEM((total, CHUNK), jnp.int32),
                pltpu.VMEM((slots, CHUNK, EMB), jnp.float32),
            ] + [pltpu.SemaphoreType.DMA] * (2 * slots),
        )
        ge, be, pe = gather(gi, bi, pi, gain_table, bw_table, pm_table)

        nb = bs // BM
        nb_off = off // BM
        if out_t is None:
            acc_ops, acc_specs, alias = [], [], {}
            body = _mlp_first
        else:
            acc_ops = [out_t]
            acc_specs = [pl.BlockSpec((LAT, BM), lambda i: (0, 0))]
            alias = {0: 0}
            body = _mlp_acc
        out_t = pl.pallas_call(
            body,
            grid=(nb,),
            in_specs=acc_specs + [row_spec, row_spec, row_spec] + w_specs,
            out_specs=pl.BlockSpec((LAT, BM),
                                   lambda i, o=nb_off: (0, o + i)),
            out_shape=jax.ShapeDtypeStruct((LAT, B), jnp.float32),
            input_output_aliases=alias,
        )(*acc_ops, ge, be, pe, W_proj, bp2, W_fc, bf2)
        off += bs
    return out_t.T
